# baseline (speedup 1.0000x reference)
import jax
import jax.numpy as jnp
from jax.experimental import pallas as pl
from jax.experimental.pallas import tpu as pltpu

_GRID = 10


def _copy_body(x_ref, e_ref, x_out, e_out):
    x_out[...] = x_ref[...]
    e_out[...] = e_ref[...]


def kernel(x, x_lstm, encoded_z_gnss, edge_index, edge_attr):
    n_nodes, d_feat = x.shape
    n_edges, d_edge = edge_attr.shape
    e_cols = 128
    e_rows = (n_edges * d_edge) // e_cols
    e_view = edge_attr.reshape(e_rows, e_cols)
    bx = n_nodes // _GRID
    be = e_rows // _GRID
    x_out, e_out = pl.pallas_call(
        _copy_body,
        grid=(_GRID,),
        out_shape=(
            jax.ShapeDtypeStruct(x.shape, x.dtype),
            jax.ShapeDtypeStruct(e_view.shape, e_view.dtype),
        ),
        in_specs=[
            pl.BlockSpec((bx, d_feat), lambda i: (i, 0)),
            pl.BlockSpec((be, e_cols), lambda i: (i, 0)),
        ],
        out_specs=(
            pl.BlockSpec((bx, d_feat), lambda i: (i, 0)),
            pl.BlockSpec((be, e_cols), lambda i: (i, 0)),
        ),
        compiler_params=pltpu.CompilerParams(
            dimension_semantics=("arbitrary",),
        ),
    )(x, e_view)
    return (x_out, e_out.reshape(n_edges, d_edge))


# speedup vs baseline: 1.0140x; 1.0140x over previous
import jax
import jax.numpy as jnp
from jax.experimental import pallas as pl
from jax.experimental.pallas import tpu as pltpu

_C = 10


def _copy_body(x_hbm, e_hbm, x_out, e_out, x_v, e_v, in_sem, out_sem):
    nx = x_hbm.shape[0] // _C
    ne = e_hbm.shape[0] // _C
    loads = []
    for i in range(_C):
        cx = pltpu.make_async_copy(
            x_hbm.at[pl.ds(i * nx, nx), :], x_v.at[pl.ds(i * nx, nx), :],
            in_sem.at[2 * i])
        ce = pltpu.make_async_copy(
            e_hbm.at[pl.ds(i * ne, ne), :], e_v.at[pl.ds(i * ne, ne), :],
            in_sem.at[2 * i + 1])
        cx.start()
        ce.start()
        loads.append((cx, ce))
    stores = []
    for i in range(_C):
        cx_in, ce_in = loads[i]
        cx_in.wait()
        ox = pltpu.make_async_copy(
            x_v.at[pl.ds(i * nx, nx), :], x_out.at[pl.ds(i * nx, nx), :],
            out_sem.at[2 * i])
        ox.start()
        ce_in.wait()
        oe = pltpu.make_async_copy(
            e_v.at[pl.ds(i * ne, ne), :], e_out.at[pl.ds(i * ne, ne), :],
            out_sem.at[2 * i + 1])
        oe.start()
        stores.append((ox, oe))
    for ox, oe in stores:
        ox.wait()
        oe.wait()


def kernel(x, x_lstm, encoded_z_gnss, edge_index, edge_attr):
    n_nodes, d_feat = x.shape
    n_edges, d_edge = edge_attr.shape
    e_cols = 128
    e_rows = (n_edges * d_edge) // e_cols
    e_view = edge_attr.reshape(e_rows, e_cols)
    x_out, e_out = pl.pallas_call(
        _copy_body,
        out_shape=(
            jax.ShapeDtypeStruct(x.shape, x.dtype),
            jax.ShapeDtypeStruct(e_view.shape, e_view.dtype),
        ),
        in_specs=[
            pl.BlockSpec(memory_space=pl.ANY),
            pl.BlockSpec(memory_space=pl.ANY),
        ],
        out_specs=(
            pl.BlockSpec(memory_space=pl.ANY),
            pl.BlockSpec(memory_space=pl.ANY),
        ),
        scratch_shapes=[
            pltpu.MemorySpace.VMEM((n_nodes, d_feat), jnp.float32),
            pltpu.MemorySpace.VMEM((e_rows, e_cols), jnp.float32),
            pltpu.SemaphoreType.DMA((2 * _C,)),
            pltpu.SemaphoreType.DMA((2 * _C,)),
        ],
    )(x, e_view)
    return (x_out, e_out.reshape(n_edges, d_edge))
